# baseline (speedup 1.0000x reference)
import functools

import jax
import jax.numpy as jnp
from jax import lax
from jax.experimental import pallas as pl
from jax.experimental.pallas import tpu as pltpu
from jax.experimental.pallas import tpu_sc as plsc

_N = 4096
_E = 131072
_NX = 512
_NH = 256
_NZ = 64
_POS_WEIGHT = float(_N * _N - _E) / _E
_NORM_LOSS = (_N * _N) / float((_N * _N - _E) * 2)

_NC = 2
_NS = 16
_NW = _NC * _NS
_C = 128
_D = 128
_EPW = _E // _NW
_T = _EPW // _C
_RPS = _N // _NS

_HIGH = jax.lax.Precision.DEFAULT


def _make_spmm(nb):
    mesh = plsc.VectorSubcoreMesh(core_axis_name="c", subcore_axis_name="s")

    nring = 3

    @functools.partial(
        pl.kernel,
        mesh=mesh,
        out_type=jax.ShapeDtypeStruct((nb, _NC, _N, _D), jnp.float32),
        scratch_types=[
            pltpu.VMEM((_C,), jnp.float32),
        ] + [
            pltpu.VMEM((_C,), jnp.int32) for _ in range(nring)
        ] + [
            pltpu.VMEM((_C,), jnp.int32) for _ in range(nring)
        ] + [
            pltpu.VMEM((_C * 16,), jnp.float32) for _ in range(nring)
        ] + [
            pltpu.VMEM((_C, _D), jnp.float32) for _ in range(nring)
        ] + [
            pltpu.VMEM_SHARED((_N, _D), jnp.float32),
            pltpu.VMEM_SHARED((_N, _D), jnp.float32),
            pltpu.SemaphoreType.DMA,
            pltpu.SemaphoreType.DMA,
        ],
    )
    def spmm(*refs):
        (h_hbm, src_hbm, dst_hbm, w_hbm, out_hbm, ws_c) = refs[:6]
        pos = 6
        srcs = refs[pos:pos + nring]; pos += nring
        dsts = refs[pos:pos + nring]; pos += nring
        wbcs = refs[pos:pos + nring]; pos += nring
        bufs = refs[pos:pos + nring]; pos += nring
        h_sh, acc_sh = refs[pos:pos + 2]; pos += 2
        sem_g, sem_s = refs[pos:pos + 2]

        c = lax.axis_index("c")
        s = lax.axis_index("s")
        wid = c * _NS + s
        eoff = wid * _EPW

        def stage_chunk(t):
            pltpu.sync_copy(dst_hbm.at[pl.ds(eoff + t * _C, _C)],
                            dsts[t % nring])
            pltpu.sync_copy(w_hbm.at[pl.ds(eoff + t * _C, _C)], ws_c)
            wbc_v = wbcs[t % nring]

            @plsc.parallel_loop(0, _C // 16, 1)
            def bc_body(g):
                wv = ws_c[pl.ds(g * 16, 16)]
                for l in range(16):
                    wbc_v[pl.ds((g * 16 + l) * 16, 16)] = jnp.broadcast_to(
                        wv[l], (16,))

        def stage_src(t):
            pltpu.sync_copy(src_hbm.at[pl.ds(eoff + t * _C, _C)],
                            srcs[t % nring])

        def gather(t):
            return pltpu.async_copy(
                h_sh.at[plsc.Indices(srcs[t % nring])],
                bufs[t % nring], sem_g)

        zv = jnp.zeros((16,), jnp.float32)

        for blk in range(nb):
            pltpu.sync_copy(h_hbm.at[blk, pl.ds(s * _RPS, _RPS)],
                            h_sh.at[pl.ds(s * _RPS, _RPS)])

            def zero_body(i, carry):
                for j in range(_D // 16):
                    bufs[0][i, pl.ds(j * 16, 16)] = zv
                return carry

            lax.fori_loop(0, _C, zero_body, 0)
            for b in range(_RPS // _C):
                pltpu.sync_copy(bufs[0],
                                acc_sh.at[pl.ds(s * _RPS + b * _C, _C)])
            plsc.subcore_barrier()

            sh = [None] * _T
            stage_src(0)
            stage_chunk(0)
            gh = gather(0)
            for t in range(_T):
                if t + 1 < _T:
                    if t >= 2:
                        sh[t - 2].wait()
                    stage_src(t + 1)
                    gh_next = gather(t + 1)
                    stage_chunk(t + 1)
                gh.wait()
                if t + 1 < _T:
                    gh = gh_next

                buf = bufs[t % nring]
                wbc_v = wbcs[t % nring]

                @plsc.parallel_loop(0, _C, 1, unroll=2)
                def mul_body(e):
                    wv16 = wbc_v[pl.ds(e * 16, 16)]
                    for j in range(_D // 16):
                        buf[e, pl.ds(j * 16, 16)] = (
                            buf[e, pl.ds(j * 16, 16)] * wv16)

                sh[t] = pltpu.async_copy(
                    buf, acc_sh.at[plsc.Indices(dsts[t % nring])],
                    sem_s, add=True)
            sh[_T - 2].wait()
            sh[_T - 1].wait()
            plsc.subcore_barrier()

            for b in range(_RPS // _C):
                r0 = s * _RPS + b * _C
                pltpu.sync_copy(acc_sh.at[pl.ds(r0, _C)], bufs[0])
                pltpu.sync_copy(bufs[0], out_hbm.at[blk, c, pl.ds(r0, _C)])
            plsc.subcore_barrier()

    return spmm


_spmm_h = _make_spmm(_NH // _D)
_spmm_z = _make_spmm(2 * _NZ // _D)


def _mm_kernel(x_ref, w_ref, o_ref):
    a = jnp.dot(x_ref[...], w_ref[...],
                precision=_HIGH, preferred_element_type=jnp.float32)
    o_ref[0] = a[:, :_D]
    o_ref[1] = a[:, _D:]


def _mid_kernel(s1_ref, w_ref, o_ref):
    h1a = jnp.maximum(s1_ref[0, 0] + s1_ref[0, 1], 0.0)
    h1b = jnp.maximum(s1_ref[1, 0] + s1_ref[1, 1], 0.0)
    o_ref[0] = (
        jnp.dot(h1a, w_ref[:_D], precision=_HIGH,
                preferred_element_type=jnp.float32)
        + jnp.dot(h1b, w_ref[_D:], precision=_HIGH,
                  preferred_element_type=jnp.float32))


def _z_kernel(s2_ref, eps_ref, z_ref, lat_ref):
    s2 = s2_ref[0, 0] + s2_ref[0, 1]
    zm = s2[:, :_NZ]
    zl = s2[:, _NZ:]
    sig = jnp.exp(zl)
    z_ref[...] = zm + eps_ref[...] * sig
    lat_ref[...] = jnp.sum(
        1.0 + 2.0 * zl - zm * zm - sig * sig).reshape(1, 1)


def _dec_kernel(zb_ref, zf_ref, lab_ref, a_ref, sum_ref):
    i = pl.program_id(0)
    a = lax.dot_general(zb_ref[...], zf_ref[...], (((1,), (1,)), ((), ())),
                        precision=_HIGH, preferred_element_type=jnp.float32)
    a_ref[...] = a
    lab = lab_ref[...]
    log_weight = 1.0 + (_POS_WEIGHT - 1.0) * lab
    ce = (1.0 - lab) * a + log_weight * (
        jnp.log1p(jnp.exp(-jnp.abs(a))) + jnp.maximum(-a, 0.0))
    part = jnp.sum(ce)

    @pl.when(i == 0)
    def _init():
        sum_ref[0, 0] = 0.0

    sum_ref[0, 0] += part


def kernel(X, edge_index, edge_weight, adj_label, eps, W1, W_mean, W_logsig):
    src = edge_index[0]
    dst = edge_index[1]
    wcat = jnp.concatenate([W_mean, W_logsig], axis=1)

    xw = pl.pallas_call(
        _mm_kernel,
        out_shape=jax.ShapeDtypeStruct((2, _N, _D), jnp.float32),
    )(X, W1)

    s1 = _spmm_h(xw, src, dst, edge_weight)

    h2 = pl.pallas_call(
        _mid_kernel,
        out_shape=jax.ShapeDtypeStruct((1, _N, _D), jnp.float32),
    )(s1, wcat)

    s2 = _spmm_z(h2, src, dst, edge_weight)

    z, lat = pl.pallas_call(
        _z_kernel,
        out_shape=(
            jax.ShapeDtypeStruct((_N, _NZ), jnp.float32),
            jax.ShapeDtypeStruct((1, 1), jnp.float32),
        ),
    )(s2, eps)

    blk = 256
    nblk = _N // blk
    a, ce_sum = pl.pallas_call(
        _dec_kernel,
        grid=(nblk,),
        in_specs=[
            pl.BlockSpec((blk, _NZ), lambda i: (i, 0)),
            pl.BlockSpec((_N, _NZ), lambda i: (0, 0)),
            pl.BlockSpec((blk, _N), lambda i: (i, 0)),
        ],
        out_specs=(
            pl.BlockSpec((blk, _N), lambda i: (i, 0)),
            pl.BlockSpec(memory_space=pltpu.SMEM, block_shape=(1, 1),
                         index_map=lambda i: (0, 0)),
        ),
        out_shape=(
            jax.ShapeDtypeStruct((_N, _N), jnp.float32),
            jax.ShapeDtypeStruct((1, 1), jnp.float32),
        ),
    )(z, z, adj_label)

    loss_latent = (-0.5 / (_N * _N)) * lat[0, 0]
    loss = _NORM_LOSS * ce_sum[0, 0] / (_N * _N) + loss_latent
    return (a, loss)


# speedup vs baseline: 1.2677x; 1.2677x over previous
import functools

import jax
import jax.numpy as jnp
from jax import lax
from jax.experimental import pallas as pl
from jax.experimental.pallas import tpu as pltpu
from jax.experimental.pallas import tpu_sc as plsc

_N = 4096
_E = 131072
_NX = 512
_NH = 256
_NZ = 64
_POS_WEIGHT = float(_N * _N - _E) / _E
_NORM_LOSS = (_N * _N) / float((_N * _N - _E) * 2)

_NC = 2
_NS = 16
_NW = _NC * _NS
_C = 128
_D = 128
_EPW = _E // _NW
_T = _EPW // _C
_RPS = _N // _NS

_HIGH = jax.lax.Precision.DEFAULT


def _make_spmm(nb):
    mesh = plsc.VectorSubcoreMesh(core_axis_name="c", subcore_axis_name="s")

    nring = 3

    @functools.partial(
        pl.kernel,
        mesh=mesh,
        out_type=jax.ShapeDtypeStruct((nb, _NC, _N, _D), jnp.float32),
        scratch_types=[
            pltpu.VMEM((3, _C), jnp.int32) for _ in range(nring)
        ] + [
            pltpu.VMEM((_C * 16,), jnp.float32) for _ in range(nring)
        ] + [
            pltpu.VMEM((_C, _D), jnp.float32) for _ in range(nring)
        ] + [
            pltpu.VMEM_SHARED((_N, _D), jnp.float32) for _ in range(nb)
        ] + [
            pltpu.SemaphoreType.DMA,
            pltpu.SemaphoreType.DMA,
        ],
    )
    def spmm(*refs):
        (h_hbm, e3_hbm, out_hbm) = refs[:3]
        pos = 3
        e3s = refs[pos:pos + nring]; pos += nring
        wbcs = refs[pos:pos + nring]; pos += nring
        bufs = refs[pos:pos + nring]; pos += nring
        accs = refs[pos:pos + nb]; pos += nb
        sem_g, sem_s = refs[pos:pos + 2]

        c = lax.axis_index("c")
        s = lax.axis_index("s")
        wid = c * _NS + s
        eoff = wid * _EPW

        def stage_src(t):
            pltpu.sync_copy(e3_hbm.at[:, pl.ds(eoff + t * _C, _C)],
                            e3s[t % nring])

        def stage_chunk(t):
            e3_v = e3s[t % nring]
            wbc_v = wbcs[t % nring]

            @plsc.parallel_loop(0, _C // 16, 1)
            def bc_body(g):
                wv = jax.lax.bitcast_convert_type(
                    e3_v[2, pl.ds(g * 16, 16)], jnp.float32)
                for l in range(16):
                    wbc_v[pl.ds((g * 16 + l) * 16, 16)] = jnp.broadcast_to(
                        wv[l], (16,))

        def gather(k):
            t, blk = divmod(k, nb)
            return pltpu.async_copy(
                h_hbm.at[blk].at[plsc.Indices(e3s[t % nring].at[0])],
                bufs[k % nring], sem_g)

        zv = jnp.zeros((16,), jnp.float32)

        def zero_body(i, carry):
            for j in range(_D // 16):
                bufs[0][i, pl.ds(j * 16, 16)] = zv
            return carry

        lax.fori_loop(0, _C, zero_body, 0)
        for acc in accs:
            for b in range(_RPS // _C):
                pltpu.sync_copy(bufs[0], acc.at[pl.ds(s * _RPS + b * _C, _C)])
        plsc.subcore_barrier()

        K = _T * nb
        sh = [None] * K
        stage_src(0)
        stage_chunk(0)
        gh = gather(0)
        for k in range(K):
            t, blk = divmod(k, nb)
            if k + 1 < K:
                tn, blkn = divmod(k + 1, nb)
                if k >= 2:
                    sh[k - 2].wait()
                if tn != t:
                    stage_src(tn)
                gh_next = gather(k + 1)
                if tn != t:
                    stage_chunk(tn)
            gh.wait()
            if k + 1 < K:
                gh = gh_next

            buf = bufs[k % nring]
            wbc_v = wbcs[t % nring]

            @plsc.parallel_loop(0, _C, 1, unroll=2)
            def mul_body(e):
                wv16 = wbc_v[pl.ds(e * 16, 16)]
                for j in range(_D // 16):
                    buf[e, pl.ds(j * 16, 16)] = (
                        buf[e, pl.ds(j * 16, 16)] * wv16)

            sh[k] = pltpu.async_copy(
                buf, accs[blk].at[plsc.Indices(e3s[t % nring].at[1])],
                sem_s, add=True)
        sh[K - 2].wait()
        sh[K - 1].wait()
        plsc.subcore_barrier()

        for blk in range(nb):
            for b in range(_RPS // _C):
                r0 = s * _RPS + b * _C
                pltpu.sync_copy(accs[blk].at[pl.ds(r0, _C)], bufs[0])
                pltpu.sync_copy(bufs[0], out_hbm.at[blk, c, pl.ds(r0, _C)])

    return spmm


_spmm_h = _make_spmm(_NH // _D)
_spmm_z = _make_spmm(2 * _NZ // _D)


def _mm_kernel(x_ref, w_ref, o_ref):
    a = jnp.dot(x_ref[...], w_ref[...],
                precision=_HIGH, preferred_element_type=jnp.float32)
    o_ref[0] = a[:, :_D]
    o_ref[1] = a[:, _D:]


def _mid_kernel(s1_ref, w_ref, o_ref):
    h1a = jnp.maximum(s1_ref[0, 0] + s1_ref[0, 1], 0.0)
    h1b = jnp.maximum(s1_ref[1, 0] + s1_ref[1, 1], 0.0)
    o_ref[0] = (
        jnp.dot(h1a, w_ref[:_D], precision=_HIGH,
                preferred_element_type=jnp.float32)
        + jnp.dot(h1b, w_ref[_D:], precision=_HIGH,
                  preferred_element_type=jnp.float32))


def _z_kernel(s2_ref, eps_ref, z_ref, lat_ref):
    s2 = s2_ref[0, 0] + s2_ref[0, 1]
    zm = s2[:, :_NZ]
    zl = s2[:, _NZ:]
    sig = jnp.exp(zl)
    z_ref[...] = zm + eps_ref[...] * sig
    lat_ref[...] = jnp.sum(
        1.0 + 2.0 * zl - zm * zm - sig * sig).reshape(1, 1)


def _dec_kernel(zb_ref, zf_ref, lab_ref, a_ref, sum_ref):
    i = pl.program_id(0)
    a = lax.dot_general(zb_ref[...], zf_ref[...], (((1,), (1,)), ((), ())),
                        precision=_HIGH, preferred_element_type=jnp.float32)
    a_ref[...] = a
    lab = lab_ref[...]
    log_weight = 1.0 + (_POS_WEIGHT - 1.0) * lab
    ce = (1.0 - lab) * a + log_weight * (
        jnp.log1p(jnp.exp(-jnp.abs(a))) + jnp.maximum(-a, 0.0))
    part = jnp.sum(ce)

    @pl.when(i == 0)
    def _init():
        sum_ref[0, 0] = 0.0

    sum_ref[0, 0] += part


def kernel(X, edge_index, edge_weight, adj_label, eps, W1, W_mean, W_logsig):
    e3 = jnp.concatenate(
        [edge_index,
         jax.lax.bitcast_convert_type(edge_weight, jnp.int32)[None]],
        axis=0)
    wcat = jnp.concatenate([W_mean, W_logsig], axis=1)

    xw = pl.pallas_call(
        _mm_kernel,
        out_shape=jax.ShapeDtypeStruct((2, _N, _D), jnp.float32),
    )(X, W1)

    s1 = _spmm_h(xw, e3)

    h2 = pl.pallas_call(
        _mid_kernel,
        out_shape=jax.ShapeDtypeStruct((1, _N, _D), jnp.float32),
    )(s1, wcat)

    s2 = _spmm_z(h2, e3)

    z, lat = pl.pallas_call(
        _z_kernel,
        out_shape=(
            jax.ShapeDtypeStruct((_N, _NZ), jnp.float32),
            jax.ShapeDtypeStruct((1, 1), jnp.float32),
        ),
    )(s2, eps)

    blk = 256
    nblk = _N // blk
    a, ce_sum = pl.pallas_call(
        _dec_kernel,
        grid=(nblk,),
        in_specs=[
            pl.BlockSpec((blk, _NZ), lambda i: (i, 0)),
            pl.BlockSpec((_N, _NZ), lambda i: (0, 0)),
            pl.BlockSpec((blk, _N), lambda i: (i, 0)),
        ],
        out_specs=(
            pl.BlockSpec((blk, _N), lambda i: (i, 0)),
            pl.BlockSpec(memory_space=pltpu.SMEM, block_shape=(1, 1),
                         index_map=lambda i: (0, 0)),
        ),
        out_shape=(
            jax.ShapeDtypeStruct((_N, _N), jnp.float32),
            jax.ShapeDtypeStruct((1, 1), jnp.float32),
        ),
    )(z, z, adj_label)

    loss_latent = (-0.5 / (_N * _N)) * lat[0, 0]
    loss = _NORM_LOSS * ce_sum[0, 0] / (_N * _N) + loss_latent
    return (a, loss)


# speedup vs baseline: 1.3184x; 1.0400x over previous
import functools

import jax
import jax.numpy as jnp
from jax import lax
from jax.experimental import pallas as pl
from jax.experimental.pallas import tpu as pltpu
from jax.experimental.pallas import tpu_sc as plsc

_N = 4096
_E = 131072
_NX = 512
_NH = 256
_NZ = 64
_POS_WEIGHT = float(_N * _N - _E) / _E
_NORM_LOSS = (_N * _N) / float((_N * _N - _E) * 2)

_NC = 2
_NS = 16
_NW = _NC * _NS
_C = 128
_D = 128
_EPW = _E // _NW
_T = _EPW // _C
_RPS = _N // _NS

_HIGH = jax.lax.Precision.DEFAULT


def _make_spmm(nb):
    mesh = plsc.VectorSubcoreMesh(core_axis_name="c", subcore_axis_name="s")

    nring = 4

    @functools.partial(
        pl.kernel,
        mesh=mesh,
        out_type=jax.ShapeDtypeStruct((nb, _NC, _N, _D), jnp.float32),
        scratch_types=[
            pltpu.VMEM((3, _C), jnp.int32) for _ in range(nring)
        ] + [
            pltpu.VMEM((_C * 16,), jnp.float32) for _ in range(nring)
        ] + [
            pltpu.VMEM((_C, _D), jnp.float32) for _ in range(nring)
        ] + [
            pltpu.VMEM_SHARED((_N, _D), jnp.float32),
            pltpu.SemaphoreType.DMA,
            pltpu.SemaphoreType.DMA,
        ],
    )
    def spmm(*refs):
        (h_hbm, e3_hbm, out_hbm) = refs[:3]
        pos = 3
        e3s = refs[pos:pos + nring]; pos += nring
        wbcs = refs[pos:pos + nring]; pos += nring
        bufs = refs[pos:pos + nring]; pos += nring
        acc_sh = refs[pos]; pos += 1
        sem_g, sem_s = refs[pos:pos + 2]

        c = lax.axis_index("c")
        s = lax.axis_index("s")
        wid = c * _NS + s
        eoff = wid * _EPW

        def stage_src(t):
            pltpu.sync_copy(e3_hbm.at[:, pl.ds(eoff + t * _C, _C)],
                            e3s[t % nring])

        def stage_chunk(t):
            e3_v = e3s[t % nring]
            wbc_v = wbcs[t % nring]

            @plsc.parallel_loop(0, _C // 16, 1)
            def bc_body(g):
                wv = jax.lax.bitcast_convert_type(
                    e3_v[2, pl.ds(g * 16, 16)], jnp.float32)
                for l in range(16):
                    wbc_v[pl.ds((g * 16 + l) * 16, 16)] = jnp.broadcast_to(
                        wv[l], (16,))

        def gather(t, blk):
            return pltpu.async_copy(
                h_hbm.at[blk].at[plsc.Indices(e3s[t % nring].at[0])],
                bufs[t % nring], sem_g)

        zv = jnp.zeros((16,), jnp.float32)

        for blk in range(nb):
            def zero_body(i, carry):
                for j in range(_D // 16):
                    bufs[0][i, pl.ds(j * 16, 16)] = zv
                return carry

            lax.fori_loop(0, _C, zero_body, 0)
            for b in range(_RPS // _C):
                pltpu.sync_copy(bufs[0],
                                acc_sh.at[pl.ds(s * _RPS + b * _C, _C)])
            plsc.subcore_barrier()

            gh = [None] * _T
            sh = [None] * _T
            stage_src(0)
            gh[0] = gather(0, blk)
            stage_chunk(0)
            stage_src(1)
            gh[1] = gather(1, blk)
            stage_chunk(1)
            for t in range(_T):
                if t + 2 < _T:
                    if t >= 2:
                        sh[t - 2].wait()
                    stage_src(t + 2)
                    gh[t + 2] = gather(t + 2, blk)
                    stage_chunk(t + 2)
                gh[t].wait()

                buf = bufs[t % nring]
                wbc_v = wbcs[t % nring]

                @plsc.parallel_loop(0, _C, 1, unroll=2)
                def mul_body(e):
                    wv16 = wbc_v[pl.ds(e * 16, 16)]
                    for j in range(_D // 16):
                        buf[e, pl.ds(j * 16, 16)] = (
                            buf[e, pl.ds(j * 16, 16)] * wv16)

                sh[t] = pltpu.async_copy(
                    buf, acc_sh.at[plsc.Indices(e3s[t % nring].at[1])],
                    sem_s, add=True)
            for tt in range(max(0, _T - 4), _T):
                sh[tt].wait()
            plsc.subcore_barrier()

            for b in range(_RPS // _C):
                r0 = s * _RPS + b * _C
                pltpu.sync_copy(acc_sh.at[pl.ds(r0, _C)], bufs[0])
                pltpu.sync_copy(bufs[0], out_hbm.at[blk, c, pl.ds(r0, _C)])
            plsc.subcore_barrier()

    return spmm


_spmm_h = _make_spmm(_NH // _D)
_spmm_z = _make_spmm(2 * _NZ // _D)


def _mm_kernel(x_ref, w_ref, o_ref):
    a = jnp.dot(x_ref[...], w_ref[...],
                precision=_HIGH, preferred_element_type=jnp.float32)
    o_ref[0] = a[:, :_D]
    o_ref[1] = a[:, _D:]


def _mid_kernel(s1_ref, w_ref, o_ref):
    h1a = jnp.maximum(s1_ref[0, 0] + s1_ref[0, 1], 0.0)
    h1b = jnp.maximum(s1_ref[1, 0] + s1_ref[1, 1], 0.0)
    o_ref[0] = (
        jnp.dot(h1a, w_ref[:_D], precision=_HIGH,
                preferred_element_type=jnp.float32)
        + jnp.dot(h1b, w_ref[_D:], precision=_HIGH,
                  preferred_element_type=jnp.float32))


def _dec_kernel(blk, s2_ref, eps_ref, lab_ref, a_ref, sum_ref, lat_ref,
                z_scr):
    i = pl.program_id(0)

    @pl.when(i == 0)
    def _init():
        s2 = s2_ref[0, 0] + s2_ref[0, 1]
        zm = s2[:, :_NZ]
        zl = s2[:, _NZ:]
        sig = jnp.exp(zl)
        z_scr[...] = zm + eps_ref[...] * sig
        lat_ref[0, 0] = jnp.sum(1.0 + 2.0 * zl - zm * zm - sig * sig)
        sum_ref[0, 0] = 0.0

    zb = z_scr[pl.ds(i * blk, blk), :]
    a = lax.dot_general(zb, z_scr[...], (((1,), (1,)), ((), ())),
                        precision=_HIGH, preferred_element_type=jnp.float32)
    a_ref[...] = a
    lab = lab_ref[...]
    log_weight = 1.0 + (_POS_WEIGHT - 1.0) * lab
    ce = (1.0 - lab) * a + log_weight * (
        jnp.log1p(jnp.exp(-jnp.abs(a))) + jnp.maximum(-a, 0.0))
    sum_ref[0, 0] += jnp.sum(ce)


def kernel(X, edge_index, edge_weight, adj_label, eps, W1, W_mean, W_logsig):
    e3 = jnp.concatenate(
        [edge_index,
         jax.lax.bitcast_convert_type(edge_weight, jnp.int32)[None]],
        axis=0)
    wcat = jnp.concatenate([W_mean, W_logsig], axis=1)

    xw = pl.pallas_call(
        _mm_kernel,
        out_shape=jax.ShapeDtypeStruct((2, _N, _D), jnp.float32),
    )(X, W1)

    s1 = _spmm_h(xw, e3)

    h2 = pl.pallas_call(
        _mid_kernel,
        out_shape=jax.ShapeDtypeStruct((1, _N, _D), jnp.float32),
    )(s1, wcat)

    s2 = _spmm_z(h2, e3)

    blk = 512
    nblk = _N // blk
    a, ce_sum, lat = pl.pallas_call(
        functools.partial(_dec_kernel, blk),
        grid=(nblk,),
        in_specs=[
            pl.BlockSpec((1, 2, _N, 2 * _NZ), lambda i: (0, 0, 0, 0)),
            pl.BlockSpec((_N, _NZ), lambda i: (0, 0)),
            pl.BlockSpec((blk, _N), lambda i: (i, 0)),
        ],
        out_specs=(
            pl.BlockSpec((blk, _N), lambda i: (i, 0)),
            pl.BlockSpec(memory_space=pltpu.SMEM, block_shape=(1, 1),
                         index_map=lambda i: (0, 0)),
            pl.BlockSpec(memory_space=pltpu.SMEM, block_shape=(1, 1),
                         index_map=lambda i: (0, 0)),
        ),
        out_shape=(
            jax.ShapeDtypeStruct((_N, _N), jnp.float32),
            jax.ShapeDtypeStruct((1, 1), jnp.float32),
            jax.ShapeDtypeStruct((1, 1), jnp.float32),
        ),
        scratch_shapes=[pltpu.VMEM((_N, _NZ), jnp.float32)],
    )(s2, eps, adj_label)

    loss_latent = (-0.5 / (_N * _N)) * lat[0, 0]
    loss = _NORM_LOSS * ce_sum[0, 0] / (_N * _N) + loss_latent
    return (a, loss)
